# Initial kernel scaffold; baseline (speedup 1.0000x reference)
import jax
import jax.numpy as jnp
from jax.experimental import pallas as pl


def kernel(x, pos, edge_index, edge_attr, Wm_0, bm_0, Wpos_0, bpos_0, Wx_0, bx_0, Wm_1, bm_1, Wpos_1, bpos_1, Wx_1, bx_1):
    raise NotImplementedError("write your pallas kernel here")


# speedup vs baseline: 2.2202x; 2.2202x over previous
import functools

import jax
import jax.numpy as jnp
from jax import lax
from jax.experimental import pallas as pl
from jax.experimental.pallas import tpu as pltpu
from jax.experimental.pallas import tpu_sc as plsc

N = 10000
E = 320000
H = 128
EA = 16
NSTEP = 64
GAMMA = 10.0
DMIN, DMAX = 0.0, 10.0

NC, NS = 2, 16
NW = NC * NS
EPW = E // NW
B = 40
NB = EPW // B
NPAD = 10240
RPT = NPAD // NS
CH = 32
XW = 256

_mesh = plsc.VectorSubcoreMesh(
    core_axis_name="c", subcore_axis_name="s", num_cores=NC, num_subcores=NS)
_sc_params = pltpu.CompilerParams(needs_layout_passes=False)


def _newton_sqrt(r2):
    i = plsc.bitcast(r2, jnp.int32)
    y = plsc.bitcast(jnp.int32(0x5F3759DF) - (i >> 1), jnp.float32)
    for _ in range(3):
        y = y * (1.5 - 0.5 * r2 * y * y)
    return jnp.where(r2 > 0.0, r2 * y, 0.0)


def _zero_shared(z_hbm, stage_v, acc, row0):
    pltpu.sync_copy(z_hbm, stage_v)
    for k in range(RPT // CH):
        pltpu.sync_copy(stage_v, acc.at[pl.ds(row0 + k * CH, CH), :])


def _read_shared(acc, stage_v, out_hbm, row0, out0):
    for k in range(RPT // CH):
        pltpu.sync_copy(acc.at[pl.ds(row0 + k * CH, CH), :], stage_v)
        pltpu.sync_copy(stage_v, out_hbm.at[pl.ds(out0 + k * CH, CH), :])


@functools.partial(
    pl.kernel,
    out_type=[jax.ShapeDtypeStruct((NC * NPAD, H), jnp.float32)],
    mesh=_mesh,
    compiler_params=_sc_params,
    scratch_types=[
        pltpu.VMEM((B,), jnp.int32),
        pltpu.VMEM((B,), jnp.int32),
        pltpu.VMEM((B, H), jnp.float32),
        pltpu.VMEM((CH, H), jnp.float32),
        pltpu.VMEM_SHARED((NPAD, H), jnp.float32),
        pltpu.SemaphoreType.DMA,
    ],
)
def _sc_gpass(x_hbm, src_hbm, dst_hbm, zg_hbm, g_out,
              src_v, dst_v, xd_v, stage_v, g_acc, sem0):
    c = lax.axis_index("c")
    s = lax.axis_index("s")
    wid = c * NS + s
    row0 = s * RPT

    _zero_shared(zg_hbm, stage_v, g_acc, row0)
    plsc.subcore_barrier()
    ebase = wid * EPW

    def it_body(it, carry):
        base = ebase + it * B
        pltpu.sync_copy(src_hbm.at[pl.ds(base, B)], src_v)
        pltpu.sync_copy(dst_hbm.at[pl.ds(base, B)], dst_v)
        pltpu.async_copy(x_hbm.at[dst_v], xd_v, sem0).wait()
        pltpu.sync_copy(xd_v, g_acc.at[src_v], add=True)
        return carry

    lax.fori_loop(0, NB, it_body, 0)
    plsc.subcore_barrier()
    _read_shared(g_acc, stage_v, g_out, row0, c * NPAD + row0)


RBF0, EA0, P40 = 0, NSTEP, NSTEP + EA


@functools.partial(
    pl.kernel,
    out_type=[jax.ShapeDtypeStruct((NC * NPAD, H), jnp.float32)],
    mesh=_mesh,
    compiler_params=_sc_params,
    scratch_types=[
        pltpu.VMEM((B,), jnp.int32),
        pltpu.VMEM((B,), jnp.int32),
        pltpu.VMEM((B, H), jnp.float32),
        pltpu.VMEM((B, XW), jnp.float32),
        pltpu.VMEM((B, EA), jnp.float32),
        pltpu.VMEM((B, H), jnp.float32),
        pltpu.VMEM((CH, H), jnp.float32),
        pltpu.VMEM_SHARED((NPAD, H), jnp.float32),
        pltpu.SemaphoreType.DMA,
        pltpu.SemaphoreType.DMA,
    ],
)
def _sc_epass(x_hbm, xe_hbm, src_hbm, dst_hbm, ea_hbm, ze_hbm, zc_hbm, e_out,
              src_v, dst_v, xs_v, xd_v, ea_v, ext2, stage_v, e_acc,
              sem0, sem1):
    c = lax.axis_index("c")
    s = lax.axis_index("s")
    wid = c * NS + s
    row0 = s * RPT

    _zero_shared(zc_hbm, stage_v, e_acc, row0)
    pltpu.sync_copy(ze_hbm, ext2)
    plsc.subcore_barrier()

    cks = [(lax.iota(jnp.int32, 16) + j * 16).astype(jnp.float32)
           * jnp.float32((DMAX - DMIN) / (NSTEP - 1)) + jnp.float32(DMIN)
           for j in range(NSTEP // 16)]
    ebase = wid * EPW

    def it_body(it, carry):
        base = ebase + it * B
        pltpu.sync_copy(src_hbm.at[pl.ds(base, B)], src_v)
        pltpu.sync_copy(dst_hbm.at[pl.ds(base, B)], dst_v)
        pltpu.sync_copy(ea_hbm.at[pl.ds(base, B), :], ea_v)
        cp0 = pltpu.async_copy(x_hbm.at[src_v], xs_v, sem0)
        cp1 = pltpu.async_copy(xe_hbm.at[dst_v], xd_v, sem1)
        cp0.wait()
        cp1.wait()

        def edge_body(e, carry2):
            p = jnp.zeros((16,), jnp.float32)
            for j in range(H // 16):
                vs = xs_v[e, pl.ds(j * 16, 16)]
                vd = xd_v[e, pl.ds(j * 16, 16)]
                t = vs - vd
                p = p + t * t
            r2 = jnp.full((16,), lax.reduce_sum_p.bind(p, axes=(0,)), jnp.float32)
            r = _newton_sqrt(r2)
            for j in range(NSTEP // 16):
                t = r - cks[j]
                ext2[e, pl.ds(RBF0 + j * 16, 16)] = jnp.exp(-GAMMA * t * t)
            ext2[e, pl.ds(EA0, EA)] = ea_v[e, :]
            ext2[e, pl.ds(P40, 16)] = xd_v[e, pl.ds(H, 16)]
            return carry2

        lax.fori_loop(0, B, edge_body, 0)
        pltpu.sync_copy(ext2, e_acc.at[src_v], add=True)
        return carry

    lax.fori_loop(0, NB, it_body, 0)
    plsc.subcore_barrier()
    _read_shared(e_acc, stage_v, e_out, row0, c * NPAD + row0)


BR = 1000


def _tc_dense_body(double_act, x_ref, g_ref, e_ref, pos_ref,
                   wm1_ref, wm2_ref, wre_ref, bm_ref, wx1_ref, wx2_ref,
                   bx_ref, xo_ref, po_ref):
    xv = x_ref[...]
    g = g_ref[0] + g_ref[1]
    ev = e_ref[0] + e_ref[1]
    p8 = ev[:, P40:P40 + 4]
    cnt = ev[:, P40 + 3:P40 + 4]
    f32 = jnp.float32
    m = jnp.dot(cnt * xv, wm1_ref[...], preferred_element_type=f32)
    m = m + jnp.dot(g, wm2_ref[...], preferred_element_type=f32)
    m = m + jnp.dot(ev, wre_ref[...], preferred_element_type=f32)
    m = m + cnt * bm_ref[...]
    h = (jnp.dot(xv, wx1_ref[...], preferred_element_type=f32)
         + jnp.dot(m, wx2_ref[...], preferred_element_type=f32) + bx_ref[...])
    h = jnp.where(h >= 0, h, 0.01 * h)
    if double_act:
        h = jnp.where(h >= 0, h, 0.01 * h)
    xo_ref[...] = h
    pos4 = pos_ref[...]
    po_ref[...] = pos4 + (cnt * pos4 - p8) / jnp.maximum(cnt, 1.0)


def _tc_dense(double_act):
    grid = (N // BR,)
    return pl.pallas_call(
        functools.partial(_tc_dense_body, double_act),
        grid=grid,
        in_specs=[
            pl.BlockSpec((BR, H), lambda i: (i, 0)),
            pl.BlockSpec((NC, BR, H), lambda i: (0, i, 0)),
            pl.BlockSpec((NC, BR, H), lambda i: (0, i, 0)),
            pl.BlockSpec((BR, 4), lambda i: (i, 0)),
            pl.BlockSpec((H, H), lambda i: (0, 0)),
            pl.BlockSpec((H, H), lambda i: (0, 0)),
            pl.BlockSpec((H, H), lambda i: (0, 0)),
            pl.BlockSpec((1, H), lambda i: (0, 0)),
            pl.BlockSpec((H, H), lambda i: (0, 0)),
            pl.BlockSpec((H, H), lambda i: (0, 0)),
            pl.BlockSpec((1, H), lambda i: (0, 0)),
        ],
        out_specs=[
            pl.BlockSpec((BR, H), lambda i: (i, 0)),
            pl.BlockSpec((BR, 4), lambda i: (i, 0)),
        ],
        out_shape=[
            jax.ShapeDtypeStruct((N, H), jnp.float32),
            jax.ShapeDtypeStruct((N, 4), jnp.float32),
        ],
    )


def kernel(x, pos, edge_index, edge_attr,
           Wm_0, bm_0, Wpos_0, bpos_0, Wx_0, bx_0,
           Wm_1, bm_1, Wpos_1, bpos_1, Wx_1, bx_1):
    src = edge_index[0]
    dst = edge_index[1]
    pos4 = jnp.concatenate([pos, jnp.ones((N, 1), jnp.float32)], axis=1)
    zg = jnp.zeros((CH, H), jnp.float32)
    ze = jnp.zeros((B, H), jnp.float32)

    xx = x
    layer_params = [(Wm_0, bm_0, Wx_0, bx_0, False), (Wm_1, bm_1, Wx_1, bx_1, True)]
    for Wm, bm, Wx, bx, double in layer_params:
        wm1 = Wm[0:H]
        wm2 = Wm[H:2 * H]
        wre = jnp.concatenate(
            [Wm[2 * H:2 * H + NSTEP + EA], jnp.zeros((H - NSTEP - EA, H), jnp.float32)],
            axis=0)
        xe = jnp.concatenate([xx, pos4, jnp.zeros((N, XW - H - 4), jnp.float32)],
                             axis=1)
        (g2,) = _sc_gpass(xx, src, dst, zg)
        (e2,) = _sc_epass(xx, xe, src, dst, edge_attr, ze, zg)
        g2 = g2.reshape(NC, NPAD, H)
        e2 = e2.reshape(NC, NPAD, H)
        xx, pos4 = _tc_dense(double)(
            xx, g2, e2, pos4,
            wm1, wm2, wre, bm.reshape(1, H),
            Wx[0:H], Wx[H:2 * H], bx.reshape(1, H))
    return (xx, pos4[:, :3])


# speedup vs baseline: 3.0053x; 1.3536x over previous
import functools

import jax
import jax.numpy as jnp
from jax import lax
from jax.experimental import pallas as pl
from jax.experimental.pallas import tpu as pltpu
from jax.experimental.pallas import tpu_sc as plsc

N = 10000
E = 320000
H = 128
EA = 16
NSTEP = 64
GAMMA = 10.0
DMIN, DMAX = 0.0, 10.0

NC, NS = 2, 16
NW = NC * NS
EPW = E // NW
B = 80
NB = EPW // B
NPAD = 10240
RPT = NPAD // NS
CH = 32
XW = 256

_mesh = plsc.VectorSubcoreMesh(
    core_axis_name="c", subcore_axis_name="s", num_cores=NC, num_subcores=NS)
_sc_params = pltpu.CompilerParams(needs_layout_passes=False)


def _newton_sqrt(r2):
    i = plsc.bitcast(r2, jnp.int32)
    y = plsc.bitcast(jnp.int32(0x5F3759DF) - (i >> 1), jnp.float32)
    for _ in range(3):
        y = y * (1.5 - 0.5 * r2 * y * y)
    return jnp.where(r2 > 0.0, r2 * y, 0.0)


def _zero_shared(z_hbm, stage_v, acc, row0):
    pltpu.sync_copy(z_hbm, stage_v)
    for k in range(RPT // CH):
        pltpu.sync_copy(stage_v, acc.at[pl.ds(row0 + k * CH, CH), :])


def _read_shared(acc, stage_v, out_hbm, row0, out0):
    for k in range(RPT // CH):
        pltpu.sync_copy(acc.at[pl.ds(row0 + k * CH, CH), :], stage_v)
        pltpu.sync_copy(stage_v, out_hbm.at[pl.ds(out0 + k * CH, CH), :])


@functools.partial(
    pl.kernel,
    out_type=[jax.ShapeDtypeStruct((NC * NPAD, H), jnp.float32)],
    mesh=_mesh,
    compiler_params=_sc_params,
    scratch_types=[
        pltpu.VMEM((B,), jnp.int32),
        pltpu.VMEM((B,), jnp.int32),
        pltpu.VMEM((B, H), jnp.float32),
        pltpu.VMEM((CH, H), jnp.float32),
        pltpu.VMEM_SHARED((NPAD, H), jnp.float32),
        pltpu.SemaphoreType.DMA,
    ],
)
def _sc_gpass(x_hbm, src_hbm, dst_hbm, zg_hbm, g_out,
              src_v, dst_v, xd_v, stage_v, g_acc, sem0):
    c = lax.axis_index("c")
    s = lax.axis_index("s")
    wid = c * NS + s
    row0 = s * RPT

    _zero_shared(zg_hbm, stage_v, g_acc, row0)
    plsc.subcore_barrier()
    ebase = wid * EPW

    def it_body(it, carry):
        base = ebase + it * B
        pltpu.sync_copy(src_hbm.at[pl.ds(base, B)], src_v)
        pltpu.sync_copy(dst_hbm.at[pl.ds(base, B)], dst_v)
        pltpu.async_copy(x_hbm.at[dst_v], xd_v, sem0).wait()
        pltpu.sync_copy(xd_v, g_acc.at[src_v], add=True)
        return carry

    lax.fori_loop(0, NB, it_body, 0)
    plsc.subcore_barrier()
    _read_shared(g_acc, stage_v, g_out, row0, c * NPAD + row0)


RBF0, EA0, P40 = 0, NSTEP, NSTEP + EA


@functools.partial(
    pl.kernel,
    out_type=[jax.ShapeDtypeStruct((NC * NPAD, H), jnp.float32)],
    mesh=_mesh,
    compiler_params=_sc_params,
    scratch_types=[
        pltpu.VMEM((B,), jnp.int32),
        pltpu.VMEM((B,), jnp.int32),
        pltpu.VMEM((B, H), jnp.float32),
        pltpu.VMEM((B, XW), jnp.float32),
        pltpu.VMEM((16, H), jnp.float32),
        pltpu.VMEM((B, H), jnp.float32),
        pltpu.VMEM((CH, H), jnp.float32),
        pltpu.VMEM_SHARED((NPAD, H), jnp.float32),
        pltpu.SemaphoreType.DMA,
        pltpu.SemaphoreType.DMA,
    ],
)
def _sc_epass(x_hbm, xe_hbm, src_hbm, dst_hbm, ea8_hbm, ze_hbm, zc_hbm, e_out,
              src_v, dst_v, xs_v, xd_v, eab, ext2, stage_v, e_acc,
              sem0, sem1):
    c = lax.axis_index("c")
    s = lax.axis_index("s")
    wid = c * NS + s
    row0 = s * RPT

    _zero_shared(zc_hbm, stage_v, e_acc, row0)
    pltpu.sync_copy(ze_hbm, ext2)
    plsc.subcore_barrier()

    cks = [(lax.iota(jnp.int32, 16) + j * 16).astype(jnp.float32)
           * jnp.float32((DMAX - DMIN) / (NSTEP - 1)) + jnp.float32(DMIN)
           for j in range(NSTEP // 16)]
    ebase = wid * EPW

    def it_body(it, carry):
        base = ebase + it * B
        pltpu.sync_copy(src_hbm.at[pl.ds(base, B)], src_v)
        pltpu.sync_copy(dst_hbm.at[pl.ds(base, B)], dst_v)
        pltpu.sync_copy(ea8_hbm.at[pl.ds((wid * NB + it) * 16, 16), :], eab)
        cp0 = pltpu.async_copy(x_hbm.at[src_v], xs_v, sem0)
        cp1 = pltpu.async_copy(xe_hbm.at[dst_v], xd_v, sem1)
        cp0.wait()
        cp1.wait()

        def blk_body(k, carry2):
            for u in range(8):
                e = k * 8 + u
                p = jnp.zeros((16,), jnp.float32)
                for j in range(H // 16):
                    vs = xs_v[e, pl.ds(j * 16, 16)]
                    vd = xd_v[e, pl.ds(j * 16, 16)]
                    t = vs - vd
                    p = p + t * t
                r2 = jnp.full((16,), lax.reduce_sum_p.bind(p, axes=(0,)),
                              jnp.float32)
                r = _newton_sqrt(r2)
                for j in range(NSTEP // 16):
                    t = r - cks[j]
                    ext2[e, pl.ds(RBF0 + j * 16, 16)] = jnp.exp(-GAMMA * t * t)
                ext2[e, pl.ds(EA0, EA)] = eab[k, pl.ds(u * EA, EA)]
                ext2[e, pl.ds(P40, 16)] = xd_v[e, pl.ds(H, 16)]
            return carry2

        lax.fori_loop(0, B // 8, blk_body, 0)
        pltpu.sync_copy(ext2, e_acc.at[src_v], add=True)
        return carry

    lax.fori_loop(0, NB, it_body, 0)
    plsc.subcore_barrier()
    _read_shared(e_acc, stage_v, e_out, row0, c * NPAD + row0)


BR = 1000


def _tc_dense_body(double_act, x_ref, g_ref, e_ref, pos_ref,
                   wm1_ref, wm2_ref, wre_ref, bm_ref, wx1_ref, wx2_ref,
                   bx_ref, xo_ref, po_ref):
    xv = x_ref[...]
    g = g_ref[0] + g_ref[1]
    ev = e_ref[0] + e_ref[1]
    p8 = ev[:, P40:P40 + 4]
    cnt = ev[:, P40 + 3:P40 + 4]
    f32 = jnp.float32
    m = jnp.dot(cnt * xv, wm1_ref[...], preferred_element_type=f32)
    m = m + jnp.dot(g, wm2_ref[...], preferred_element_type=f32)
    m = m + jnp.dot(ev, wre_ref[...], preferred_element_type=f32)
    m = m + cnt * bm_ref[...]
    h = (jnp.dot(xv, wx1_ref[...], preferred_element_type=f32)
         + jnp.dot(m, wx2_ref[...], preferred_element_type=f32) + bx_ref[...])
    h = jnp.where(h >= 0, h, 0.01 * h)
    if double_act:
        h = jnp.where(h >= 0, h, 0.01 * h)
    xo_ref[...] = h
    pos4 = pos_ref[...]
    po_ref[...] = pos4 + (cnt * pos4 - p8) / jnp.maximum(cnt, 1.0)


def _tc_dense(double_act):
    grid = (N // BR,)
    return pl.pallas_call(
        functools.partial(_tc_dense_body, double_act),
        grid=grid,
        in_specs=[
            pl.BlockSpec((BR, H), lambda i: (i, 0)),
            pl.BlockSpec((NC, BR, H), lambda i: (0, i, 0)),
            pl.BlockSpec((NC, BR, H), lambda i: (0, i, 0)),
            pl.BlockSpec((BR, 4), lambda i: (i, 0)),
            pl.BlockSpec((H, H), lambda i: (0, 0)),
            pl.BlockSpec((H, H), lambda i: (0, 0)),
            pl.BlockSpec((H, H), lambda i: (0, 0)),
            pl.BlockSpec((1, H), lambda i: (0, 0)),
            pl.BlockSpec((H, H), lambda i: (0, 0)),
            pl.BlockSpec((H, H), lambda i: (0, 0)),
            pl.BlockSpec((1, H), lambda i: (0, 0)),
        ],
        out_specs=[
            pl.BlockSpec((BR, H), lambda i: (i, 0)),
            pl.BlockSpec((BR, 4), lambda i: (i, 0)),
        ],
        out_shape=[
            jax.ShapeDtypeStruct((N, H), jnp.float32),
            jax.ShapeDtypeStruct((N, 4), jnp.float32),
        ],
    )


def kernel(x, pos, edge_index, edge_attr,
           Wm_0, bm_0, Wpos_0, bpos_0, Wx_0, bx_0,
           Wm_1, bm_1, Wpos_1, bpos_1, Wx_1, bx_1):
    src = edge_index[0]
    dst = edge_index[1]
    pos4 = jnp.concatenate([pos, jnp.ones((N, 1), jnp.float32)], axis=1)
    ea8 = jnp.pad(edge_attr.reshape(NW * NB, B // 8, 8 * EA),
                  ((0, 0), (0, 16 - B // 8), (0, 0))).reshape(-1, 8 * EA)
    zg = jnp.zeros((CH, H), jnp.float32)
    ze = jnp.zeros((B, H), jnp.float32)

    xx = x
    layer_params = [(Wm_0, bm_0, Wx_0, bx_0, False), (Wm_1, bm_1, Wx_1, bx_1, True)]
    for Wm, bm, Wx, bx, double in layer_params:
        wm1 = Wm[0:H]
        wm2 = Wm[H:2 * H]
        wre = jnp.concatenate(
            [Wm[2 * H:2 * H + NSTEP + EA], jnp.zeros((H - NSTEP - EA, H), jnp.float32)],
            axis=0)
        xe = jnp.concatenate([xx, pos4, jnp.zeros((N, XW - H - 4), jnp.float32)],
                             axis=1)
        (g2,) = _sc_gpass(xx, src, dst, zg)
        (e2,) = _sc_epass(xx, xe, src, dst, ea8, ze, zg)
        g2 = g2.reshape(NC, NPAD, H)
        e2 = e2.reshape(NC, NPAD, H)
        xx, pos4 = _tc_dense(double)(
            xx, g2, e2, pos4,
            wm1, wm2, wre, bm.reshape(1, H),
            Wx[0:H], Wx[H:2 * H], bx.reshape(1, H))
    return (xx, pos4[:, :3])
